# baseline (speedup 1.0000x reference)
import functools

import jax
import jax.numpy as jnp
from jax import lax
from jax.experimental import pallas as pl
from jax.experimental.pallas import tpu as pltpu
from jax.experimental.pallas import tpu_sc as plsc

N = 10000
E = 320000
D = 128
H = 128
NS = 100

NC = 2
NSC = 16
NW = NC * NSC
CH = 112
NCH = 90
NPAIR = NCH // 2
EPW = NCH * CH
EPAD = NW * EPW
NCH0 = 106
NCH1 = 74
EPW0 = NCH0 * CH
EPW1 = NCH1 * CH
NP = 10112
NPT = NP // NSC

_MESH = dict(core_axis_name="c", subcore_axis_name="s", num_cores=NC,
             num_subcores=NSC)


@functools.partial(
    pl.kernel,
    out_type=jax.ShapeDtypeStruct((NC, NP, H), jnp.float32),
    mesh=plsc.VectorSubcoreMesh(**_MESH),
    scratch_types=[
        pltpu.VMEM((NCH, CH), jnp.int32),
        pltpu.VMEM((CH, H), jnp.float32),
        pltpu.VMEM_SHARED((NP, H), jnp.float32),
        pltpu.SemaphoreType.DMA,
    ],
)
def _sc_degree(dst3_hbm, ones_hbm, zeros_hbm, out_hbm, didx_v, ones_v,
               deg_sh, sem):
    c = lax.axis_index("c")
    s = lax.axis_index("s")
    wid = s * NC + c
    pltpu.sync_copy(ones_hbm, ones_v)
    pltpu.sync_copy(dst3_hbm.at[wid], didx_v)
    pltpu.sync_copy(zeros_hbm.at[pl.ds(s * NPT, NPT)],
                    deg_sh.at[pl.ds(s * NPT, NPT)])
    plsc.subcore_barrier()

    def group(g, carry):
        def fire(i, carry):
            pltpu.async_copy(ones_v, deg_sh.at[didx_v.at[g * 6 + i]], sem,
                             add=True)
            return carry
        lax.fori_loop(0, 6, fire, 0)
        def drain(i, carry):
            pltpu.make_async_copy(ones_hbm, ones_v, sem).wait()
            return carry
        lax.fori_loop(0, 6, drain, 0)
        return carry

    lax.fori_loop(0, NCH // 6, group, 0)
    plsc.subcore_barrier()
    pltpu.sync_copy(deg_sh.at[pl.ds(s * NPT, NPT)],
                    out_hbm.at[c, pl.ds(s * NPT, NPT)])


@functools.partial(
    pl.kernel,
    out_type=jax.ShapeDtypeStruct((NC, NP, H), jnp.float32),
    mesh=plsc.VectorSubcoreMesh(**_MESH),
    scratch_types=[
        pltpu.VMEM((CH,), jnp.int32),
        pltpu.VMEM((CH,), jnp.int32),
        pltpu.VMEM((CH,), jnp.int32),
        pltpu.VMEM((CH,), jnp.int32),
        pltpu.VMEM((CH, H), jnp.float32),
        pltpu.VMEM((CH, H), jnp.float32),
        pltpu.VMEM_SHARED((NP, H), jnp.float32),
        pltpu.SemaphoreType.DMA,
        pltpu.SemaphoreType.DMA,
    ],
)
def _sc_propagate(table_hbm, src_hbm, dst_hbm, zeros_hbm, out_hbm,
                  sidx0, didx0, sidx1, didx1, rows0, rows1, acc_sh,
                  sem0, sem1):
    c = lax.axis_index("c")
    s = lax.axis_index("s")
    pltpu.sync_copy(zeros_hbm.at[pl.ds(s * NPT, NPT)],
                    acc_sh.at[pl.ds(s * NPT, NPT)])
    plsc.subcore_barrier()

    def run(base, npair):
        pltpu.sync_copy(src_hbm.at[pl.ds(base, CH)], sidx0)
        pltpu.sync_copy(dst_hbm.at[pl.ds(base, CH)], didx0)
        pltpu.async_copy(table_hbm.at[sidx0], rows0, sem0)

        def body(g, carry):
            off1 = base + (2 * g + 1) * CH
            pltpu.sync_copy(src_hbm.at[pl.ds(off1, CH)], sidx1)
            pltpu.sync_copy(dst_hbm.at[pl.ds(off1, CH)], didx1)
            pltpu.async_copy(table_hbm.at[sidx1], rows1, sem1)

            pltpu.make_async_copy(table_hbm.at[sidx0], rows0, sem0).wait()
            pltpu.sync_copy(rows0, acc_sh.at[didx0], add=True)

            @pl.when(g + 1 < npair)
            def _():
                off2 = base + (2 * g + 2) * CH
                pltpu.sync_copy(src_hbm.at[pl.ds(off2, CH)], sidx0)
                pltpu.sync_copy(dst_hbm.at[pl.ds(off2, CH)], didx0)
                pltpu.async_copy(table_hbm.at[sidx0], rows0, sem0)

            pltpu.make_async_copy(table_hbm.at[sidx1], rows1, sem1).wait()
            pltpu.sync_copy(rows1, acc_sh.at[didx1], add=True)
            return carry

        lax.fori_loop(0, npair, body, 0)

    @pl.when(c == 0)
    def _():
        run(s * EPW0, NCH0 // 2)

    @pl.when(c == 1)
    def _():
        run(NSC * EPW0 + s * EPW1, NCH1 // 2)

    plsc.subcore_barrier()
    pltpu.sync_copy(acc_sh.at[pl.ds(s * NPT, NPT)],
                    out_hbm.at[c, pl.ds(s * NPT, NPT)])


GRID_R = 10
BR = N // GRID_R


def _dinv_block(degp_ref):
    deg = degp_ref[0][:, 0:1] + degp_ref[1][:, 0:1] + 1.0
    return lax.rsqrt(deg)


def _mm1_body(degp_ref, x_ref, w1_ref, o_ref):
    dinv = _dinv_block(degp_ref)
    o_ref[...] = dinv * jnp.dot(x_ref[...], w1_ref[...],
                                preferred_element_type=jnp.float32)


def _mm2_body(degp_ref, p_ref, s1_ref, b1_ref, w2_ref, o_ref):
    dinv = _dinv_block(degp_ref)
    t = p_ref[0] + p_ref[1] + s1_ref[...]
    h1 = jnp.maximum(dinv * t + b1_ref[...], 0.0)
    o_ref[...] = dinv * jnp.dot(h1, w2_ref[...],
                                preferred_element_type=jnp.float32)


def _mm3_body(degp_ref, p_ref, s2_ref, b2_ref, o_ref):
    dinv = _dinv_block(degp_ref)
    o_ref[...] = dinv * (p_ref[0] + p_ref[1] + s2_ref[...]) + b2_ref[...]


def _mm4_body(f_ref, wa_ref, ba_ref, o_ref):
    k = pl.program_id(0)

    @pl.when(k == 0)
    def _():
        o_ref[...] = jnp.zeros_like(o_ref) + ba_ref[...]

    o_ref[...] += jnp.dot(f_ref[...], wa_ref[...],
                          preferred_element_type=jnp.float32)


def _mm1(degp, x, W1):
    return pl.pallas_call(
        _mm1_body,
        grid=(GRID_R,),
        in_specs=[
            pl.BlockSpec((NC, BR, H), lambda i: (0, i, 0)),
            pl.BlockSpec((BR, D), lambda i: (i, 0)),
            pl.BlockSpec((D, H), lambda i: (0, 0)),
        ],
        out_specs=pl.BlockSpec((BR, H), lambda i: (i, 0)),
        out_shape=jax.ShapeDtypeStruct((N, H), jnp.float32),
    )(degp, x, W1)


def _mm2(degp, parts, S1, b1, W2):
    return pl.pallas_call(
        _mm2_body,
        grid=(GRID_R,),
        in_specs=[
            pl.BlockSpec((NC, BR, H), lambda i: (0, i, 0)),
            pl.BlockSpec((NC, BR, H), lambda i: (0, i, 0)),
            pl.BlockSpec((BR, H), lambda i: (i, 0)),
            pl.BlockSpec((1, H), lambda i: (0, 0)),
            pl.BlockSpec((H, H), lambda i: (0, 0)),
        ],
        out_specs=pl.BlockSpec((BR, H), lambda i: (i, 0)),
        out_shape=jax.ShapeDtypeStruct((N, H), jnp.float32),
    )(degp, parts, S1, b1, W2)


def _mm3(degp, parts, S2, b2):
    return pl.pallas_call(
        _mm3_body,
        grid=(GRID_R,),
        in_specs=[
            pl.BlockSpec((NC, BR, H), lambda i: (0, i, 0)),
            pl.BlockSpec((NC, BR, H), lambda i: (0, i, 0)),
            pl.BlockSpec((BR, H), lambda i: (i, 0)),
            pl.BlockSpec((1, H), lambda i: (0, 0)),
        ],
        out_specs=pl.BlockSpec((BR, H), lambda i: (i, 0)),
        out_shape=jax.ShapeDtypeStruct((N, H), jnp.float32),
    )(degp, parts, S2, b2)


GRID_K = 10
BK = NS * H // GRID_K


def _mm4(flat, Wa, ba):
    return pl.pallas_call(
        _mm4_body,
        grid=(GRID_K,),
        in_specs=[
            pl.BlockSpec((N // NS, BK), lambda k: (0, k)),
            pl.BlockSpec((BK, H), lambda k: (k, 0)),
            pl.BlockSpec((1, H), lambda k: (0, 0)),
        ],
        out_specs=pl.BlockSpec((N // NS, H), lambda k: (0, 0)),
        out_shape=jax.ShapeDtypeStruct((N // NS, H), jnp.float32),
    )(flat, Wa, ba)


def kernel(x, edge_index, W1, b1, W2, b2, Wa, ba):
    pad = EPAD - E
    src = jnp.concatenate([edge_index[0], jnp.zeros((pad,), jnp.int32)])
    dst = jnp.concatenate([edge_index[1], jnp.full((pad,), N, jnp.int32)])
    zeros_nh = jnp.zeros((NP, H), jnp.float32)
    ones_ch = jnp.ones((CH, H), jnp.float32)

    degp = _sc_degree(dst.reshape(NW, NCH, CH), ones_ch, zeros_nh)
    S1 = _mm1(degp, x, W1)
    P1 = _sc_propagate(S1, src, dst, zeros_nh)
    S2 = _mm2(degp, P1, S1, b1.reshape(1, H), W2)
    P2 = _sc_propagate(S2, src, dst, zeros_nh)
    emb = _mm3(degp, P2, S2, b2.reshape(1, H))
    flat = emb.reshape(N // NS, NS * H)
    cell = _mm4(flat, Wa, ba.reshape(1, H))
    return (cell, emb)


# speedup vs baseline: 1.0246x; 1.0246x over previous
import functools

import jax
import jax.numpy as jnp
from jax import lax
from jax.experimental import pallas as pl
from jax.experimental.pallas import tpu as pltpu
from jax.experimental.pallas import tpu_sc as plsc

N = 10000
E = 320000
D = 128
H = 128
NS = 100

NC = 2
NSC = 16
NW = NC * NSC
CH = 112
NCH = 90
NPAIR = NCH // 2
EPW = NCH * CH
EPAD = NW * EPW
NCH0 = 112
NCH1 = 68
EPW0 = NCH0 * CH
EPW1 = NCH1 * CH
NP = 10112
NPT = NP // NSC

_MESH = dict(core_axis_name="c", subcore_axis_name="s", num_cores=NC,
             num_subcores=NSC)


@functools.partial(
    pl.kernel,
    out_type=jax.ShapeDtypeStruct((NC, NP, H), jnp.float32),
    mesh=plsc.VectorSubcoreMesh(**_MESH),
    scratch_types=[
        pltpu.VMEM((NCH, CH), jnp.int32),
        pltpu.VMEM((CH, H), jnp.float32),
        pltpu.VMEM_SHARED((NP, H), jnp.float32),
        pltpu.SemaphoreType.DMA,
    ],
)
def _sc_degree(dst3_hbm, ones_hbm, zeros_hbm, out_hbm, didx_v, ones_v,
               deg_sh, sem):
    c = lax.axis_index("c")
    s = lax.axis_index("s")
    wid = s * NC + c
    pltpu.sync_copy(ones_hbm, ones_v)
    pltpu.sync_copy(dst3_hbm.at[wid], didx_v)
    pltpu.sync_copy(zeros_hbm.at[pl.ds(s * NPT, NPT)],
                    deg_sh.at[pl.ds(s * NPT, NPT)])
    plsc.subcore_barrier()

    def group(g, carry):
        def fire(i, carry):
            pltpu.async_copy(ones_v, deg_sh.at[didx_v.at[g * 6 + i]], sem,
                             add=True)
            return carry
        lax.fori_loop(0, 6, fire, 0)
        def drain(i, carry):
            pltpu.make_async_copy(ones_hbm, ones_v, sem).wait()
            return carry
        lax.fori_loop(0, 6, drain, 0)
        return carry

    lax.fori_loop(0, NCH // 6, group, 0)
    plsc.subcore_barrier()
    pltpu.sync_copy(deg_sh.at[pl.ds(s * NPT, NPT)],
                    out_hbm.at[c, pl.ds(s * NPT, NPT)])


@functools.partial(
    pl.kernel,
    out_type=jax.ShapeDtypeStruct((NC, NP, H), jnp.float32),
    mesh=plsc.VectorSubcoreMesh(**_MESH),
    scratch_types=[
        pltpu.VMEM((CH,), jnp.int32),
        pltpu.VMEM((CH,), jnp.int32),
        pltpu.VMEM((CH,), jnp.int32),
        pltpu.VMEM((CH,), jnp.int32),
        pltpu.VMEM((CH, H), jnp.float32),
        pltpu.VMEM((CH, H), jnp.float32),
        pltpu.VMEM_SHARED((NP, H), jnp.float32),
        pltpu.SemaphoreType.DMA,
        pltpu.SemaphoreType.DMA,
    ],
)
def _sc_propagate(table_hbm, src_hbm, dst_hbm, zeros_hbm, out_hbm,
                  sidx0, didx0, sidx1, didx1, rows0, rows1, acc_sh,
                  sem0, sem1):
    c = lax.axis_index("c")
    s = lax.axis_index("s")
    pltpu.sync_copy(zeros_hbm.at[pl.ds(s * NPT, NPT)],
                    acc_sh.at[pl.ds(s * NPT, NPT)])
    plsc.subcore_barrier()

    def run(base, npair):
        pltpu.sync_copy(src_hbm.at[pl.ds(base, CH)], sidx0)
        pltpu.sync_copy(dst_hbm.at[pl.ds(base, CH)], didx0)
        pltpu.async_copy(table_hbm.at[sidx0], rows0, sem0)

        def body(g, carry):
            off1 = base + (2 * g + 1) * CH
            pltpu.sync_copy(src_hbm.at[pl.ds(off1, CH)], sidx1)
            pltpu.sync_copy(dst_hbm.at[pl.ds(off1, CH)], didx1)
            pltpu.async_copy(table_hbm.at[sidx1], rows1, sem1)

            pltpu.make_async_copy(table_hbm.at[sidx0], rows0, sem0).wait()
            pltpu.sync_copy(rows0, acc_sh.at[didx0], add=True)

            @pl.when(g + 1 < npair)
            def _():
                off2 = base + (2 * g + 2) * CH
                pltpu.sync_copy(src_hbm.at[pl.ds(off2, CH)], sidx0)
                pltpu.sync_copy(dst_hbm.at[pl.ds(off2, CH)], didx0)
                pltpu.async_copy(table_hbm.at[sidx0], rows0, sem0)

            pltpu.make_async_copy(table_hbm.at[sidx1], rows1, sem1).wait()
            pltpu.sync_copy(rows1, acc_sh.at[didx1], add=True)
            return carry

        lax.fori_loop(0, npair, body, 0)

    @pl.when(c == 0)
    def _():
        run(s * EPW0, NCH0 // 2)

    @pl.when(c == 1)
    def _():
        run(NSC * EPW0 + s * EPW1, NCH1 // 2)

    plsc.subcore_barrier()
    pltpu.sync_copy(acc_sh.at[pl.ds(s * NPT, NPT)],
                    out_hbm.at[c, pl.ds(s * NPT, NPT)])


GRID_R = 10
BR = N // GRID_R


def _dinv_block(degp_ref):
    deg = degp_ref[0][:, 0:1] + degp_ref[1][:, 0:1] + 1.0
    return lax.rsqrt(deg)


def _mm1_body(degp_ref, x_ref, w1_ref, o_ref):
    dinv = _dinv_block(degp_ref)
    o_ref[...] = dinv * jnp.dot(x_ref[...], w1_ref[...],
                                preferred_element_type=jnp.float32)


def _mm2_body(degp_ref, p_ref, s1_ref, b1_ref, w2_ref, o_ref):
    dinv = _dinv_block(degp_ref)
    t = p_ref[0] + p_ref[1] + s1_ref[...]
    h1 = jnp.maximum(dinv * t + b1_ref[...], 0.0)
    o_ref[...] = dinv * jnp.dot(h1, w2_ref[...],
                                preferred_element_type=jnp.float32)


def _mm3_body(degp_ref, p_ref, s2_ref, b2_ref, o_ref):
    dinv = _dinv_block(degp_ref)
    o_ref[...] = dinv * (p_ref[0] + p_ref[1] + s2_ref[...]) + b2_ref[...]


def _mm4_body(f_ref, wa_ref, ba_ref, o_ref):
    k = pl.program_id(0)

    @pl.when(k == 0)
    def _():
        o_ref[...] = jnp.zeros_like(o_ref) + ba_ref[...]

    o_ref[...] += jnp.dot(f_ref[...], wa_ref[...],
                          preferred_element_type=jnp.float32)


def _mm1(degp, x, W1):
    return pl.pallas_call(
        _mm1_body,
        grid=(GRID_R,),
        in_specs=[
            pl.BlockSpec((NC, BR, H), lambda i: (0, i, 0)),
            pl.BlockSpec((BR, D), lambda i: (i, 0)),
            pl.BlockSpec((D, H), lambda i: (0, 0)),
        ],
        out_specs=pl.BlockSpec((BR, H), lambda i: (i, 0)),
        out_shape=jax.ShapeDtypeStruct((N, H), jnp.float32),
    )(degp, x, W1)


def _mm2(degp, parts, S1, b1, W2):
    return pl.pallas_call(
        _mm2_body,
        grid=(GRID_R,),
        in_specs=[
            pl.BlockSpec((NC, BR, H), lambda i: (0, i, 0)),
            pl.BlockSpec((NC, BR, H), lambda i: (0, i, 0)),
            pl.BlockSpec((BR, H), lambda i: (i, 0)),
            pl.BlockSpec((1, H), lambda i: (0, 0)),
            pl.BlockSpec((H, H), lambda i: (0, 0)),
        ],
        out_specs=pl.BlockSpec((BR, H), lambda i: (i, 0)),
        out_shape=jax.ShapeDtypeStruct((N, H), jnp.float32),
    )(degp, parts, S1, b1, W2)


def _mm3(degp, parts, S2, b2):
    return pl.pallas_call(
        _mm3_body,
        grid=(GRID_R,),
        in_specs=[
            pl.BlockSpec((NC, BR, H), lambda i: (0, i, 0)),
            pl.BlockSpec((NC, BR, H), lambda i: (0, i, 0)),
            pl.BlockSpec((BR, H), lambda i: (i, 0)),
            pl.BlockSpec((1, H), lambda i: (0, 0)),
        ],
        out_specs=pl.BlockSpec((BR, H), lambda i: (i, 0)),
        out_shape=jax.ShapeDtypeStruct((N, H), jnp.float32),
    )(degp, parts, S2, b2)


GRID_K = 10
BK = NS * H // GRID_K


def _mm4(flat, Wa, ba):
    return pl.pallas_call(
        _mm4_body,
        grid=(GRID_K,),
        in_specs=[
            pl.BlockSpec((N // NS, BK), lambda k: (0, k)),
            pl.BlockSpec((BK, H), lambda k: (k, 0)),
            pl.BlockSpec((1, H), lambda k: (0, 0)),
        ],
        out_specs=pl.BlockSpec((N // NS, H), lambda k: (0, 0)),
        out_shape=jax.ShapeDtypeStruct((N // NS, H), jnp.float32),
    )(flat, Wa, ba)


def kernel(x, edge_index, W1, b1, W2, b2, Wa, ba):
    pad = EPAD - E
    src = jnp.concatenate([edge_index[0], jnp.zeros((pad,), jnp.int32)])
    dst = jnp.concatenate([edge_index[1], jnp.full((pad,), N, jnp.int32)])
    zeros_nh = jnp.zeros((NP, H), jnp.float32)
    ones_ch = jnp.ones((CH, H), jnp.float32)

    degp = _sc_degree(dst.reshape(NW, NCH, CH), ones_ch, zeros_nh)
    S1 = _mm1(degp, x, W1)
    P1 = _sc_propagate(S1, src, dst, zeros_nh)
    S2 = _mm2(degp, P1, S1, b1.reshape(1, H), W2)
    P2 = _sc_propagate(S2, src, dst, zeros_nh)
    emb = _mm3(degp, P2, S2, b2.reshape(1, H))
    flat = emb.reshape(N // NS, NS * H)
    cell = _mm4(flat, Wa, ba.reshape(1, H))
    return (cell, emb)
